# Initial kernel scaffold; baseline (speedup 1.0000x reference)
import jax
import jax.numpy as jnp
from jax.experimental import pallas as pl


def kernel(X, Y, W, rows, cols):
    raise NotImplementedError("write your pallas kernel here")


# speedup vs baseline: 1.9572x; 1.9572x over previous
import functools
import math

import jax
import jax.numpy as jnp
from jax import lax
from jax.experimental import pallas as pl
from jax.experimental.pallas import tpu as pltpu
from jax.experimental.pallas import tpu_sc as plsc

NW = 32
WND = 128
C0 = 1.0 / math.sqrt(2.0)
C0D = 1.0 / math.sqrt(6.0)
C1 = 1.0 / math.sqrt(3.0)
C1C = 1.0 / math.sqrt(6.0)


def _block_plan(chunk):
    b = 200 if chunk % 200 == 0 else chunk
    while chunk % b or b % 8:
        b -= 8
    halves = []
    off = 0
    while b - off > 128:
        halves.append((off, 96))
        off += 96
    halves.append((off, b - off))
    return b, tuple(halves)


def _make_sc_main(n_nodes, n_edges):
    chunk = n_edges // NW
    B, halves = _block_plan(chunk)
    nblk = chunk // B
    npair = (nblk - 1) // 2

    mesh = plsc.VectorSubcoreMesh(core_axis_name="c", subcore_axis_name="s",
                                  num_cores=2, num_subcores=16)
    scratch = []
    for _ in range(2):
        for (_, hn) in halves:
            scratch.append(pltpu.VMEM((hn,), jnp.int32))
        scratch.append(pltpu.VMEM((B + 16,), jnp.int32))
        scratch.append(pltpu.VMEM((B * 4 + 16,), jnp.float32))
        scratch.append(pltpu.VMEM((B * 80,), jnp.float32))
        for (_, hn) in halves:
            scratch.append(pltpu.VMEM((hn, 128), jnp.float32))
    scratch += [
        pltpu.VMEM((WND * 64,), jnp.float32),
        pltpu.VMEM((128,), jnp.float32),
        pltpu.VMEM((16,), jnp.int32),
        pltpu.VMEM((16,), jnp.int32),
        pltpu.SemaphoreType.DMA,
        pltpu.SemaphoreType.DMA,
        pltpu.SemaphoreType.DMA,
        pltpu.SemaphoreType.DMA,
    ]
    nh = len(halves)
    per_slot = nh + 3 + nh

    @functools.partial(
        pl.kernel,
        out_type=(
            jax.ShapeDtypeStruct((n_nodes * 64 + 64,), jnp.float32),
            jax.ShapeDtypeStruct((NW * 128,), jnp.float32),
            jax.ShapeDtypeStruct((NW * 16,), jnp.int32),
        ),
        mesh=mesh,
        scratch_types=scratch,
        compiler_params=pltpu.CompilerParams(needs_layout_passes=False),
    )
    def sc_main(x_hbm, yf_hbm, wf_hbm, rows_hbm, cols_hbm, z_hbm, bout_hbm,
                brow_hbm, *scr):
        slots = []
        for si in range(2):
            g0 = si * per_slot
            slots.append(dict(
                cols=scr[g0:g0 + nh],
                rows=scr[g0 + nh],
                y=scr[g0 + nh + 1],
                w=scr[g0 + nh + 2],
                x=scr[g0 + nh + 3:g0 + nh + 3 + nh],
            ))
        winbuf = scr[2 * per_slot]
        bbuf = scr[2 * per_slot + 1]
        rbuf = scr[2 * per_slot + 2]
        endbuf = scr[2 * per_slot + 3]
        sem_a = scr[2 * per_slot + 4:2 * per_slot + 6]
        sem_g = scr[2 * per_slot + 6:2 * per_slot + 8]

        wid = lax.axis_index("c") * 16 + lax.axis_index("s")
        e0 = wid * chunk
        lanes = lax.iota(jnp.int32, 16)
        zero16 = jnp.zeros((16,), jnp.float32)

        def splat(v):
            return jnp.full((16,), v, jnp.int32)

        def a_descs(si, g):
            base = e0 + g * B
            s = slots[si]
            d = []
            for hi, (ho, hn) in enumerate(halves):
                d.append(pltpu.make_async_copy(
                    cols_hbm.at[pl.ds(base + ho, hn)], s["cols"][hi],
                    sem_a[si]))
            d.append(pltpu.make_async_copy(
                rows_hbm.at[pl.ds(base, B)], s["rows"].at[pl.ds(0, B)],
                sem_a[si]))
            d.append(pltpu.make_async_copy(
                yf_hbm.at[pl.ds(base * 4, B * 4)],
                s["y"].at[pl.ds(0, B * 4)], sem_a[si]))
            d.append(pltpu.make_async_copy(
                wf_hbm.at[pl.ds(base * 80, B * 80)], s["w"], sem_a[si]))
            return d

        def g_descs(si):
            s = slots[si]
            return [
                pltpu.make_async_copy(x_hbm.at[s["cols"][hi]], s["x"][hi],
                                      sem_g[si])
                for hi in range(nh)
            ]

        def start(descs):
            for d in descs:
                d.start()

        def wait(descs):
            for d in descs:
                d.wait()

        def zero_window():
            def zr(j, _):
                winbuf[pl.ds(16 * j, 16)] = zero16
                return 0
            lax.fori_loop(0, WND * 4, zr, 0)

        def drain(_, wb):
            pltpu.sync_copy(winbuf, z_hbm.at[pl.ds(wb * 64, WND * 64)])
            zero_window()
            return wb + WND

        def write_bentry(slot, row, a0, ax, ay, az):
            off = slot * 64
            bbuf[pl.ds(off + 0, 16)] = a0
            bbuf[pl.ds(off + 16, 16)] = ax
            bbuf[pl.ds(off + 32, 16)] = ay
            bbuf[pl.ds(off + 48, 16)] = az
            rbuf[...] = jnp.where(lanes == slot, splat(row), rbuf[...])

        def do_flush(cr, fl, wb, a0, ax, ay, az):
            def first(cr, wb, a0, ax, ay, az):
                write_bentry(0, cr, a0, ax, ay, az)
                return wb

            def interior(cr, wb, a0, ax, ay, az):
                nadv = lax.shift_right_logical(cr - wb, 7)
                wb = lax.fori_loop(0, nadv, drain, wb)
                base = (cr - wb) * 64
                winbuf[pl.ds(base, 16)] = a0
                winbuf[pl.ds(base + 16, 16)] = ax
                winbuf[pl.ds(base + 32, 16)] = ay
                winbuf[pl.ds(base + 48, 16)] = az
                return wb

            wb = lax.cond(fl == 0, first, interior, cr, wb, a0, ax, ay, az)
            return jnp.int32(1), wb

        def make_edge_body(s, hi, roff):
            rows_b, y_b, w_b = s["rows"], s["y"], s["w"]
            x_b = s["x"][hi]

            def edge_body(i, st):
                cur_row, fl, wb, a0, ax, ay, az = st
                e = roff + i
                r = rows_b[pl.ds(e, 16)][0]
                ch = r != cur_row
                fl, wb = lax.cond(
                    ch, do_flush,
                    lambda cr, fl, wb, *_: (fl, wb),
                    cur_row, fl, wb, a0, ax, ay, az)
                x0 = x_b[i, pl.ds(0, 16)]
                x1x = x_b[i, pl.ds(16, 16)]
                x1y = x_b[i, pl.ds(32, 16)]
                x1z = x_b[i, pl.ds(48, 16)]
                wo = e * 80
                w1 = w_b[pl.ds(wo, 16)]
                w2 = w_b[pl.ds(wo + 16, 16)]
                w3 = w_b[pl.ds(wo + 32, 16)]
                w4 = w_b[pl.ds(wo + 48, 16)]
                w5 = w_b[pl.ds(wo + 64, 16)]
                yv = y_b[pl.ds(e * 4, 16)]
                y0 = jnp.full((16,), yv[0], jnp.float32)
                yx = jnp.full((16,), yv[1], jnp.float32)
                yy = jnp.full((16,), yv[2], jnp.float32)
                yz = jnp.full((16,), yv[3], jnp.float32)
                dot = x1x * yx + x1y * yy + x1z * yz
                o0 = C0 * (w1 * x0 * y0) + C0D * (w4 * dot)
                crx = x1y * yz - x1z * yy
                cry = x1z * yx - x1x * yz
                crz = x1x * yy - x1y * yx
                ox = C1 * (w2 * x1x * y0 + w3 * x0 * yx) + C1C * (w5 * crx)
                oy = C1 * (w2 * x1y * y0 + w3 * x0 * yy) + C1C * (w5 * cry)
                oz = C1 * (w2 * x1z * y0 + w3 * x0 * yz) + C1C * (w5 * crz)
                keep = jnp.full((16,), jnp.where(ch, 0.0, 1.0), jnp.float32)
                return (r, fl, wb,
                        a0 * keep + o0, ax * keep + ox,
                        ay * keep + oy, az * keep + oz)

            return edge_body

        def process(g, si, st):
            wait(g_descs(si))

            @pl.when(g + 1 < nblk)
            def _():
                wait(a_descs(1 - si, g + 1))
                start(g_descs(1 - si))

            s = slots[si]
            for hi, (ho, hn) in enumerate(halves):
                st = lax.fori_loop(0, hn, make_edge_body(s, hi, ho), st)

            @pl.when(g + 2 < nblk)
            def _():
                start(a_descs(si, g + 2))

            return st

        start(a_descs(0, 0))
        wait(a_descs(0, 0))
        start(g_descs(0))

        @pl.when(nblk > 1)
        def _():
            start(a_descs(1, 1))

        endbuf[...] = jnp.zeros((16,), jnp.int32)

        @pl.when(wid < NW - 1)
        def _():
            pltpu.sync_copy(rows_hbm.at[pl.ds((wid + 1) * chunk, 16)], endbuf)

        end_t = jnp.where(wid == NW - 1, jnp.int32(n_nodes), endbuf[...][0])
        zero_window()
        row0 = slots[0]["rows"][pl.ds(0, 16)][0]
        rbuf[...] = jnp.zeros((16,), jnp.int32)
        write_bentry(0, row0, zero16, zero16, zero16, zero16)
        win0 = jnp.where(wid == 0, jnp.int32(0), row0)
        st = (row0, jnp.int32(0), win0, zero16, zero16, zero16, zero16)

        def pair_body(gg, st):
            for si in range(2):
                st = process(2 * gg + si, si, st)
            return st

        st = lax.fori_loop(0, npair, pair_body, st)
        for g in range(2 * npair, nblk):
            st = process(g, g & 1, st)
        cur_row, fl, wb = st[0], st[1], st[2]

        write_bentry(1, cur_row, st[3], st[4], st[5], st[6])
        ntail = lax.shift_right_logical(end_t - wb, 7)
        wb = lax.fori_loop(0, ntail, drain, wb)
        rem = end_t - wb
        soff = jnp.int32(0)
        for bit in (64, 32, 16, 8, 4, 2, 1):
            p = (rem & bit) != 0

            @pl.when(p)
            def _(soff=soff, bit=bit):
                pltpu.sync_copy(winbuf.at[pl.ds(soff * 64, bit * 64)],
                                z_hbm.at[pl.ds((wb + soff) * 64, bit * 64)])

            soff = soff + bit * p.astype(jnp.int32)
        pltpu.sync_copy(bbuf, bout_hbm.at[pl.ds(wid * 128, 128)])
        pltpu.sync_copy(rbuf, brow_hbm.at[pl.ds(wid * 16, 16)])

    return sc_main


def _combine_body(z_in, bv, rc, rr, brs, z_out, nbuf, wbuf, sem):
    n = bv.shape[0]

    def span(i):
        return pl.multiple_of((brs[i] >> 1) * 128, 128)

    for i in range(n):
        pltpu.make_async_copy(z_in.at[pl.ds(span(i), 128)],
                              nbuf.at[i], sem).start()
    for i in range(n):
        pltpu.make_async_copy(z_in.at[pl.ds(span(i), 128)],
                              nbuf.at[i], sem).wait()
    mf = (rc[...] == rr[...]).astype(jnp.float32)
    comb = jnp.dot(mf, bv[...], preferred_element_type=jnp.float32)
    m2 = (rr[...] == (rc[...] ^ 1)).astype(jnp.float32)
    cnt = jnp.sum(m2, axis=1, keepdims=True)
    p = jnp.dot(m2, comb, preferred_element_type=jnp.float32)
    parity = (rc[...] & 1) == 1
    other_read = jnp.where(parity, nbuf[:, 0:64], nbuf[:, 64:128])
    other = jnp.where(cnt > 0, p / jnp.maximum(cnt, 1.0), other_read)
    left = jnp.where(parity, other, comb)
    right = jnp.where(parity, comb, other)
    wbuf[...] = jnp.concatenate([left, right], axis=1)
    for i in range(n):
        pltpu.make_async_copy(wbuf.at[i],
                              z_out.at[pl.ds(span(i), 128)], sem).start()
    for i in range(n):
        pltpu.make_async_copy(wbuf.at[i],
                              z_out.at[pl.ds(span(i), 128)], sem).wait()


def _combine(zp, bv, rc, rr, brs):
    return pl.pallas_call(
        _combine_body,
        out_shape=jax.ShapeDtypeStruct(zp.shape, zp.dtype),
        in_specs=[
            pl.BlockSpec(memory_space=pl.ANY),
            pl.BlockSpec(memory_space=pltpu.MemorySpace.VMEM),
            pl.BlockSpec(memory_space=pltpu.MemorySpace.VMEM),
            pl.BlockSpec(memory_space=pltpu.MemorySpace.VMEM),
            pl.BlockSpec(memory_space=pltpu.MemorySpace.SMEM),
        ],
        out_specs=pl.BlockSpec(memory_space=pl.ANY),
        input_output_aliases={0: 0},
        scratch_shapes=[pltpu.VMEM((NW * 2, 128), jnp.float32),
                        pltpu.VMEM((NW * 2, 128), jnp.float32),
                        pltpu.SemaphoreType.DMA],
    )(zp, bv, rc, rr, brs)


@functools.lru_cache(maxsize=None)
def _build(n_nodes, n_edges):
    sc_main = _make_sc_main(n_nodes, n_edges)

    @jax.jit
    def run(x, y, w, rows, cols):
        xg = jnp.concatenate(
            [x[:, :16],
             x[:, 16:].reshape(n_nodes, 16, 3).transpose(0, 2, 1)
             .reshape(n_nodes, 48),
             jnp.zeros((n_nodes, 64), jnp.float32)], axis=1)
        zp, bout, brow = sc_main(xg, y.reshape(-1), w.reshape(-1), rows, cols)
        bvals = bout.reshape(NW * 2, 64)
        brows = brow.reshape(NW, 16)[:, :2].reshape(-1)
        zc = _combine(zp, bvals, brows.reshape(-1, 1),
                      brows.reshape(1, -1), brows)
        zc = zc[:n_nodes * 64].reshape(n_nodes, 64)
        z1 = zc[:, 16:].reshape(n_nodes, 3, 16).transpose(0, 2, 1)
        return jnp.concatenate([zc[:, :16], z1.reshape(n_nodes, 48)], axis=1)

    return run


def kernel(X, Y, W, rows, cols):
    return _build(X.shape[0], rows.shape[0])(X, Y, W, rows, cols)
